# baseline (device time: 29359 ns/iter reference)
import jax
import jax.numpy as jnp
from jax import lax
from jax.experimental import pallas as pl
from jax.experimental.pallas import tpu as pltpu

N_DEV = 4


def kernel(q, k, v):
    s_per, d = q.shape
    scale = 1.0 / (d ** 0.5)

    def body(q_ref, k_ref, v_ref, out_ref, kc_ref, vc_ref,
             ksend, krecv, vsend, vrecv):
        my = lax.axis_index("i")
        left = (my - 1) % N_DEV
        right = (my + 1) % N_DEV

        barrier_sem = pltpu.get_barrier_semaphore()
        for nbr in [left, right]:
            pl.semaphore_signal(
                barrier_sem, inc=1,
                device_id=(nbr,), device_id_type=pl.DeviceIdType.MESH,
            )
        pl.semaphore_wait(barrier_sem, 2)

        q_val = q_ref[:, :]

        m = jnp.full((s_per, 1), -jnp.inf, dtype=jnp.float32)
        l = jnp.zeros((s_per, 1), dtype=jnp.float32)
        acc = jnp.zeros((s_per, d), dtype=jnp.float32)

        def attn_update(m, l, acc, k_c, v_c):
            s = lax.dot_general(
                q_val, k_c, (((1,), (1,)), ((), ())),
                preferred_element_type=jnp.float32,
            ) * scale
            m_new = jnp.maximum(m, jnp.max(s, axis=1, keepdims=True))
            alpha = jnp.exp(m - m_new)
            p = jnp.exp(s - m_new)
            l = l * alpha + jnp.sum(p, axis=1, keepdims=True)
            acc = acc * alpha + lax.dot(
                p, v_c, preferred_element_type=jnp.float32
            )
            return m_new, l, acc

        for h in range(N_DEV):
            if h == 0:
                k_src, v_src = k_ref, v_ref
            else:
                k_src, v_src = kc_ref.at[h - 1], vc_ref.at[h - 1]

            if h < N_DEV - 1:
                k_rdma = pltpu.make_async_remote_copy(
                    src_ref=k_src, dst_ref=kc_ref.at[h],
                    send_sem=ksend.at[h], recv_sem=krecv.at[h],
                    device_id=(right,), device_id_type=pl.DeviceIdType.MESH,
                )
                v_rdma = pltpu.make_async_remote_copy(
                    src_ref=v_src, dst_ref=vc_ref.at[h],
                    send_sem=vsend.at[h], recv_sem=vrecv.at[h],
                    device_id=(right,), device_id_type=pl.DeviceIdType.MESH,
                )
                k_rdma.start()
                v_rdma.start()

            m, l, acc = attn_update(m, l, acc, k_src[...], v_src[...])

            if h < N_DEV - 1:
                k_rdma.wait()
                v_rdma.wait()

        out_ref[:, :] = acc / l

    return pl.pallas_call(
        body,
        out_shape=jax.ShapeDtypeStruct((s_per, d), jnp.float32),
        in_specs=[
            pl.BlockSpec(memory_space=pltpu.VMEM),
            pl.BlockSpec(memory_space=pltpu.VMEM),
            pl.BlockSpec(memory_space=pltpu.VMEM),
        ],
        out_specs=pl.BlockSpec(memory_space=pltpu.VMEM),
        scratch_shapes=[
            pltpu.VMEM((N_DEV - 1, s_per, d), jnp.float32),
            pltpu.VMEM((N_DEV - 1, s_per, d), jnp.float32),
            pltpu.SemaphoreType.DMA((N_DEV - 1,)),
            pltpu.SemaphoreType.DMA((N_DEV - 1,)),
            pltpu.SemaphoreType.DMA((N_DEV - 1,)),
            pltpu.SemaphoreType.DMA((N_DEV - 1,)),
        ],
        compiler_params=pltpu.CompilerParams(collective_id=0),
    )(q, k, v)


# device time: 15014 ns/iter; 1.9554x vs baseline; 1.9554x over previous
import jax
import jax.numpy as jnp
from jax import lax
from jax.experimental import pallas as pl
from jax.experimental.pallas import tpu as pltpu

N_DEV = 4


def kernel(q, k, v):
    s_per, d = q.shape
    scale = 1.0 / (d ** 0.5)

    def body(q_ref, k_ref, v_ref, out_ref,
             kloc, vloc, kbuf, vbuf, ksend, krecv, vsend, vrecv):
        my = lax.axis_index("i")

        kloc[:, :] = k_ref[:, :].astype(jnp.bfloat16)
        vloc[:, :] = v_ref[:, :].astype(jnp.bfloat16)
        q_bf = q_ref[:, :].astype(jnp.bfloat16)

        barrier_sem = pltpu.get_barrier_semaphore()
        for off in range(1, N_DEV):
            pl.semaphore_signal(
                barrier_sem, inc=1,
                device_id=((my + off) % N_DEV,),
                device_id_type=pl.DeviceIdType.MESH,
            )
        pl.semaphore_wait(barrier_sem, N_DEV - 1)

        rdmas = []
        for off in (1, 3, 2):
            tgt = (my + off) % N_DEV
            k_rdma = pltpu.make_async_remote_copy(
                src_ref=kloc, dst_ref=kbuf.at[off - 1],
                send_sem=ksend.at[off - 1], recv_sem=krecv.at[off - 1],
                device_id=(tgt,), device_id_type=pl.DeviceIdType.MESH,
            )
            v_rdma = pltpu.make_async_remote_copy(
                src_ref=vloc, dst_ref=vbuf.at[off - 1],
                send_sem=vsend.at[off - 1], recv_sem=vrecv.at[off - 1],
                device_id=(tgt,), device_id_type=pl.DeviceIdType.MESH,
            )
            k_rdma.start()
            v_rdma.start()
            rdmas.append((k_rdma, v_rdma))

        m = jnp.full((s_per, 1), -jnp.inf, dtype=jnp.float32)
        l = jnp.zeros((s_per, 1), dtype=jnp.float32)
        acc = jnp.zeros((s_per, d), dtype=jnp.float32)

        def attn_update(m, l, acc, k_c, v_c):
            s = lax.dot_general(
                q_bf, k_c, (((1,), (1,)), ((), ())),
                preferred_element_type=jnp.float32,
            ) * scale
            m_new = jnp.maximum(m, jnp.max(s, axis=1, keepdims=True))
            alpha = jnp.exp(m - m_new)
            p = jnp.exp(s - m_new)
            l = l * alpha + jnp.sum(p, axis=1, keepdims=True)
            acc = acc * alpha + lax.dot(
                p.astype(jnp.bfloat16), v_c,
                preferred_element_type=jnp.float32,
            )
            return m_new, l, acc

        m, l, acc = attn_update(m, l, acc, kloc[...], vloc[...])

        for slot in (0, 2, 1):
            k_rdma, v_rdma = rdmas[(1, 3, 2).index(slot + 1)]
            k_rdma.wait()
            v_rdma.wait()
            m, l, acc = attn_update(m, l, acc, kbuf[slot], vbuf[slot])

        out_ref[:, :] = acc / l

    return pl.pallas_call(
        body,
        out_shape=jax.ShapeDtypeStruct((s_per, d), jnp.float32),
        in_specs=[
            pl.BlockSpec(memory_space=pltpu.VMEM),
            pl.BlockSpec(memory_space=pltpu.VMEM),
            pl.BlockSpec(memory_space=pltpu.VMEM),
        ],
        out_specs=pl.BlockSpec(memory_space=pltpu.VMEM),
        scratch_shapes=[
            pltpu.VMEM((s_per, d), jnp.bfloat16),
            pltpu.VMEM((s_per, d), jnp.bfloat16),
            pltpu.VMEM((N_DEV - 1, s_per, d), jnp.bfloat16),
            pltpu.VMEM((N_DEV - 1, s_per, d), jnp.bfloat16),
            pltpu.SemaphoreType.DMA((N_DEV - 1,)),
            pltpu.SemaphoreType.DMA((N_DEV - 1,)),
            pltpu.SemaphoreType.DMA((N_DEV - 1,)),
            pltpu.SemaphoreType.DMA((N_DEV - 1,)),
        ],
        compiler_params=pltpu.CompilerParams(collective_id=0),
    )(q, k, v)
